# baseline (speedup 1.0000x reference)
import jax
import jax.numpy as jnp
from jax.experimental import pallas as pl
from jax.experimental.pallas import tpu as pltpu

_B, _N, _H, _K = 16, 1024, 64, 15
_T = 64
_ALPHA = 0.3
_NEG = -1000000000.0
_KNOCK = -2000000000.0


def _embed(ctl, drug, wge_ref, bge_ref):
    ctl_c = jnp.transpose(ctl)
    drug_c = jnp.transpose(drug)
    wge = wge_ref[...]
    return jnp.maximum(ctl_c * wge[0:1, :] + drug_c * wge[1:2, :]
                       + bge_ref[...], 0.0)


def _head(x2, ctl, cemb_ref, w1_ref, bf1_ref, w2_ref, bf2_ref):
    cemb = cemb_ref[...].reshape(1, _H)
    w1 = w1_ref[...]
    h = jnp.maximum(
        jnp.dot(x2, w1[0:_H, :], preferred_element_type=jnp.float32)
        + jnp.dot(cemb, w1[_H:, :], preferred_element_type=jnp.float32)
        + bf1_ref[...], 0.0)
    pred_c = jnp.sum(h * w2_ref[...], axis=1, keepdims=True)
    pred = jnp.transpose(pred_c)
    return (ctl + pred + bf2_ref[0, 0]).reshape(1, 1, _N)


def _fast_layer(adj_bf, x, attn_row, kern, bias_row):
    imp_col = jnp.sum(x * attn_row, axis=1, keepdims=True)
    imp_row = jnp.transpose(imp_col)
    support = jnp.dot(x, kern, preferred_element_type=jnp.float32)

    gt_f = (imp_col > imp_row).astype(jnp.float32)
    eq_f = (imp_col == imp_row).astype(jnp.float32)
    ones_row = jnp.ones((1, _N), jnp.float32)
    rank_row = jnp.dot(ones_row, gt_f,
                       preferred_element_type=jnp.float32
                       ).astype(jnp.int32)
    rank_col = jnp.transpose(rank_row)
    eq_total = jnp.sum(jnp.dot(ones_row, eq_f,
                               preferred_element_type=jnp.float32))
    tie_flag = jnp.where(eq_total > _N + 0.5, 1.0, 0.0)

    iota_tn = jax.lax.broadcasted_iota(jnp.int32, (_T, _N), 0)
    iota_nt = jax.lax.broadcasted_iota(jnp.int32, (_N, _T), 1)
    p_tn = iota_tn == rank_row
    pt_nt = rank_col == iota_nt
    sv_row = jnp.max(jnp.where(pt_nt, jnp.broadcast_to(imp_col, (_N, _T)),
                               _KNOCK), axis=0, keepdims=True)

    adjc = jnp.dot(adj_bf, pt_nt.astype(jnp.float32).astype(jnp.bfloat16),
                   preferred_element_type=jnp.float32)
    maskc = jnp.where(adjc > 0.0, 1.0, 0.0)
    tri_u = (jax.lax.broadcasted_iota(jnp.int32, (_T, _T), 0)
             <= jax.lax.broadcasted_iota(jnp.int32, (_T, _T), 1)
             ).astype(jnp.float32).astype(jnp.bfloat16)
    cum = jnp.dot(maskc.astype(jnp.bfloat16), tri_u,
                  preferred_element_type=jnp.float32)
    sel = (maskc > 0.5) & (cum < _K + 0.5)
    vt = cum[:, _T - 1:_T]
    flag = jnp.maximum(
        jnp.max(jnp.where(vt < _K - 0.5, 1.0, 0.0)), tie_flag)

    m_sel = jnp.max(jnp.where(sel, jnp.broadcast_to(sv_row, (_N, _T)),
                              _KNOCK), axis=1, keepdims=True)
    w = jnp.where(sel, jnp.exp(sv_row - m_sel), 0.0)
    denom = jnp.sum(w, axis=1, keepdims=True)
    support_cand = jnp.dot(p_tn.astype(jnp.float32), support,
                           preferred_element_type=jnp.float32)
    agg = jnp.dot(w, support_cand,
                  preferred_element_type=jnp.float32) / denom
    out = jnp.maximum(_ALPHA * x + (1.0 - _ALPHA) * agg + bias_row, 0.0)
    return out, flag


def _fast_body(adj_ref, ctl_ref, drug_ref, cemb_ref, wge_ref, bge_ref,
               a1_ref, k1_ref, b1_ref, a2_ref, k2_ref, b2_ref,
               w1_ref, bf1_ref, w2_ref, bf2_ref, out_ref, flag_ref):
    adj_bf = adj_ref[...].astype(jnp.bfloat16)
    ctl = ctl_ref[...].reshape(1, _N)
    drug = drug_ref[...].reshape(1, _N)
    x0 = _embed(ctl, drug, wge_ref, bge_ref)
    x1, f1 = _fast_layer(adj_bf, x0, a1_ref[...], k1_ref[...], b1_ref[...])
    x2, f2 = _fast_layer(adj_bf, x1, a2_ref[...], k2_ref[...], b2_ref[...])
    out_ref[...] = _head(x2, ctl, cemb_ref, w1_ref, bf1_ref, w2_ref, bf2_ref)
    flag_ref[...] = jnp.broadcast_to(jnp.maximum(f1, f2), (1, 1, 128))


def _slow_agg(mask, imp_row, support):
    vals = jnp.where(mask, jnp.broadcast_to(imp_row, (_N, _N)), _NEG)
    iota = jax.lax.broadcasted_iota(jnp.int32, (_N, _N), 1)
    dmat = jnp.zeros((_N, _N), jnp.float32)
    denom = jnp.zeros((_N, 1), jnp.float32)
    v1 = None
    for k in range(_K):
        m = jnp.max(vals, axis=1, keepdims=True)
        selc = jnp.min(jnp.where(vals == m, iota, _N), axis=1,
                       keepdims=True)
        onehot = iota == selc
        if k == 0:
            v1 = m
            e = jnp.ones((_N, 1), jnp.float32)
        else:
            e = jnp.exp(m - v1)
        dmat = jnp.where(onehot, jnp.broadcast_to(e, (_N, _N)), dmat)
        denom = denom + e
        vals = jnp.where(onehot, _KNOCK, vals)
    return jnp.dot(dmat, support, preferred_element_type=jnp.float32) / denom


def _slow_layer(mask, x, attn_row, kern, bias_row):
    imp_col = jnp.sum(x * attn_row, axis=1, keepdims=True)
    imp_row = jnp.transpose(imp_col)
    support = jnp.dot(x, kern, preferred_element_type=jnp.float32)
    agg = _slow_agg(mask, imp_row, support)
    return jnp.maximum(_ALPHA * x + (1.0 - _ALPHA) * agg + bias_row, 0.0)


def _slow_body(adj_ref, ctl_ref, drug_ref, cemb_ref, wge_ref, bge_ref,
               a1_ref, k1_ref, b1_ref, a2_ref, k2_ref, b2_ref,
               w1_ref, bf1_ref, w2_ref, bf2_ref, out_ref):
    mask = adj_ref[...] > 0.0
    ctl = ctl_ref[...].reshape(1, _N)
    drug = drug_ref[...].reshape(1, _N)
    x0 = _embed(ctl, drug, wge_ref, bge_ref)
    x1 = _slow_layer(mask, x0, a1_ref[...], k1_ref[...], b1_ref[...])
    x2 = _slow_layer(mask, x1, a2_ref[...], k2_ref[...], b2_ref[...])
    out_ref[...] = _head(x2, ctl, cemb_ref, w1_ref, bf1_ref, w2_ref, bf2_ref)


def _in_specs():
    full2 = lambda shape: pl.BlockSpec(shape, lambda b: (0, 0))
    per_b = lambda shape: pl.BlockSpec(shape, lambda b: (b, 0, 0))
    return [
        full2((_N, _N)),
        per_b((1, 1, _N)),
        per_b((1, 1, _N)),
        per_b((1, 1, _H)),
        full2((2, _H)),
        full2((1, _H)),
        full2((1, _H)),
        full2((_H, _H)),
        full2((1, _H)),
        full2((1, _H)),
        full2((_H, _H)),
        full2((1, _H)),
        full2((2 * _H, _H)),
        full2((1, _H)),
        full2((1, _H)),
        full2((1, 1)),
    ], per_b


def _run_fast(args, interpret=False):
    in_specs, per_b = _in_specs()
    return pl.pallas_call(
        _fast_body,
        grid=(_B,),
        in_specs=in_specs,
        out_specs=[per_b((1, 1, _N)), per_b((1, 1, 128))],
        out_shape=[jax.ShapeDtypeStruct((_B, 1, _N), jnp.float32),
                   jax.ShapeDtypeStruct((_B, 1, 128), jnp.float32)],
        compiler_params=pltpu.CompilerParams(
            dimension_semantics=("parallel",)),
        interpret=interpret,
    )(*args)


def _run_slow(args, interpret=False):
    in_specs, per_b = _in_specs()
    return pl.pallas_call(
        _slow_body,
        grid=(_B,),
        in_specs=in_specs,
        out_specs=per_b((1, 1, _N)),
        out_shape=jax.ShapeDtypeStruct((_B, 1, _N), jnp.float32),
        compiler_params=pltpu.CompilerParams(
            dimension_semantics=("parallel",)),
        interpret=interpret,
    )(*args)


def kernel(adj, ctl_expr, drug_targets, cell_idx, W_ge, b_ge, cell_table,
           attn1, kernel1, bias1, attn2, kernel2, bias2, W1, b1, W2, b2):
    cemb = jnp.take(cell_table, cell_idx, axis=0)
    args = (
        adj,
        ctl_expr.reshape(_B, 1, _N),
        drug_targets.reshape(_B, 1, _N),
        cemb.reshape(_B, 1, _H),
        W_ge,
        b_ge.reshape(1, _H),
        attn1.reshape(1, _H),
        kernel1,
        bias1.reshape(1, _H),
        attn2.reshape(1, _H),
        kernel2,
        bias2.reshape(1, _H),
        W1,
        b1.reshape(1, _H),
        W2.reshape(1, _H),
        b2.reshape(1, 1),
    )
    out_fast, flags = _run_fast(args)
    any_deficient = jnp.sum(flags[:, 0, 0]) > 0.0
    out = jax.lax.cond(
        any_deficient,
        lambda: _run_slow(args),
        lambda: out_fast,
    )
    return out.reshape(_B, _N)


# speedup vs baseline: 9.7323x; 9.7323x over previous
import jax
import jax.numpy as jnp
from jax.experimental import pallas as pl
from jax.experimental.pallas import tpu as pltpu

_B, _N, _H, _K = 16, 1024, 64, 15
_T = 64
_ALPHA = 0.3
_NEG = -1000000000.0
_KNOCK = -2000000000.0


def _embed(ctl, drug, wge_ref, bge_ref):
    ctl_c = jnp.transpose(ctl)
    drug_c = jnp.transpose(drug)
    wge = wge_ref[...]
    return jnp.maximum(ctl_c * wge[0:1, :] + drug_c * wge[1:2, :]
                       + bge_ref[...], 0.0)


def _head(x2, ctl, cemb_ref, w1_ref, bf1_ref, w2_ref, bf2_ref):
    cemb = cemb_ref[...].reshape(1, _H)
    w1 = w1_ref[...]
    h = jnp.maximum(
        jnp.dot(x2, w1[0:_H, :], preferred_element_type=jnp.float32)
        + jnp.dot(cemb, w1[_H:, :], preferred_element_type=jnp.float32)
        + bf1_ref[...], 0.0)
    pred_c = jnp.sum(h * w2_ref[...], axis=1, keepdims=True)
    pred = jnp.transpose(pred_c)
    return (ctl + pred + bf2_ref[0, 0]).reshape(1, 1, _N)


def _fast_layer(adj_bf, x, attn_row, kern, bias_row):
    imp_col = jnp.sum(x * attn_row, axis=1, keepdims=True)
    imp_row = jnp.transpose(imp_col)
    support = jnp.dot(x, kern, preferred_element_type=jnp.float32)

    gt_f = (imp_col > imp_row).astype(jnp.float32)
    ones_row = jnp.ones((1, _N), jnp.float32)
    rank_row = jnp.dot(ones_row, gt_f,
                       preferred_element_type=jnp.float32
                       ).astype(jnp.int32)
    rank_col = jnp.transpose(rank_row)

    iota_tn = jax.lax.broadcasted_iota(jnp.int32, (_T, _N), 0)
    iota_nt = jax.lax.broadcasted_iota(jnp.int32, (_N, _T), 1)
    p_tn = (iota_tn == rank_row).astype(jnp.float32)
    pt_nt = rank_col == iota_nt
    tie_flag = jnp.max(jnp.where(
        jnp.abs(jnp.sum(p_tn, axis=1, keepdims=True) - 1.0) > 0.5,
        1.0, 0.0))
    sv_row = jnp.max(jnp.where(pt_nt, jnp.broadcast_to(imp_col, (_N, _T)),
                               _KNOCK), axis=0, keepdims=True)


    adjc = jnp.dot(adj_bf, pt_nt.astype(jnp.float32).astype(jnp.bfloat16),
                   preferred_element_type=jnp.float32)
    maskc = jnp.where(adjc > 0.0, 1.0, 0.0)
    tri_u = (jax.lax.broadcasted_iota(jnp.int32, (_T, _T), 0)
             <= jax.lax.broadcasted_iota(jnp.int32, (_T, _T), 1)
             ).astype(jnp.float32).astype(jnp.bfloat16)
    cum = jnp.dot(maskc.astype(jnp.bfloat16), tri_u,
                  preferred_element_type=jnp.float32)
    sel = (maskc > 0.5) & (cum < _K + 0.5)
    vt = cum[:, _T - 1:_T]
    flag = jnp.maximum(
        jnp.max(jnp.where(vt < _K - 0.5, 1.0, 0.0)), tie_flag)

    m_sel = jnp.max(jnp.where(sel, jnp.broadcast_to(sv_row, (_N, _T)),
                              _KNOCK), axis=1, keepdims=True)
    w = jnp.where(sel, jnp.exp(sv_row - m_sel), 0.0)
    denom = jnp.sum(w, axis=1, keepdims=True)
    support_cand = jnp.dot(p_tn, support,
                           preferred_element_type=jnp.float32)
    agg = jnp.dot(w, support_cand,
                  preferred_element_type=jnp.float32) / denom
    out = jnp.maximum(_ALPHA * x + (1.0 - _ALPHA) * agg + bias_row, 0.0)
    return out, flag


def _fast_body(adj_ref, ctl_ref, drug_ref, cemb_ref, wge_ref, bge_ref,
               a1_ref, k1_ref, b1_ref, a2_ref, k2_ref, b2_ref,
               w1_ref, bf1_ref, w2_ref, bf2_ref, out_ref, flag_ref):
    adj_bf = adj_ref[...].astype(jnp.bfloat16)
    ctl = ctl_ref[...].reshape(1, _N)
    drug = drug_ref[...].reshape(1, _N)
    x0 = _embed(ctl, drug, wge_ref, bge_ref)
    x1, f1 = _fast_layer(adj_bf, x0, a1_ref[...], k1_ref[...], b1_ref[...])
    x2, f2 = _fast_layer(adj_bf, x1, a2_ref[...], k2_ref[...], b2_ref[...])
    out_ref[...] = _head(x2, ctl, cemb_ref, w1_ref, bf1_ref, w2_ref, bf2_ref)
    flag_ref[...] = jnp.broadcast_to(jnp.maximum(f1, f2), (1, 1, 128))


def _slow_agg(mask, imp_row, support):
    vals = jnp.where(mask, jnp.broadcast_to(imp_row, (_N, _N)), _NEG)
    iota = jax.lax.broadcasted_iota(jnp.int32, (_N, _N), 1)
    dmat = jnp.zeros((_N, _N), jnp.float32)
    denom = jnp.zeros((_N, 1), jnp.float32)
    v1 = None
    for k in range(_K):
        m = jnp.max(vals, axis=1, keepdims=True)
        selc = jnp.min(jnp.where(vals == m, iota, _N), axis=1,
                       keepdims=True)
        onehot = iota == selc
        if k == 0:
            v1 = m
            e = jnp.ones((_N, 1), jnp.float32)
        else:
            e = jnp.exp(m - v1)
        dmat = jnp.where(onehot, jnp.broadcast_to(e, (_N, _N)), dmat)
        denom = denom + e
        vals = jnp.where(onehot, _KNOCK, vals)
    return jnp.dot(dmat, support, preferred_element_type=jnp.float32) / denom


def _slow_layer(mask, x, attn_row, kern, bias_row):
    imp_col = jnp.sum(x * attn_row, axis=1, keepdims=True)
    imp_row = jnp.transpose(imp_col)
    support = jnp.dot(x, kern, preferred_element_type=jnp.float32)
    agg = _slow_agg(mask, imp_row, support)
    return jnp.maximum(_ALPHA * x + (1.0 - _ALPHA) * agg + bias_row, 0.0)


def _slow_body(adj_ref, ctl_ref, drug_ref, cemb_ref, wge_ref, bge_ref,
               a1_ref, k1_ref, b1_ref, a2_ref, k2_ref, b2_ref,
               w1_ref, bf1_ref, w2_ref, bf2_ref, out_ref):
    mask = adj_ref[...] > 0.0
    ctl = ctl_ref[...].reshape(1, _N)
    drug = drug_ref[...].reshape(1, _N)
    x0 = _embed(ctl, drug, wge_ref, bge_ref)
    x1 = _slow_layer(mask, x0, a1_ref[...], k1_ref[...], b1_ref[...])
    x2 = _slow_layer(mask, x1, a2_ref[...], k2_ref[...], b2_ref[...])
    out_ref[...] = _head(x2, ctl, cemb_ref, w1_ref, bf1_ref, w2_ref, bf2_ref)


def _in_specs():
    full2 = lambda shape: pl.BlockSpec(shape, lambda b: (0, 0))
    per_b = lambda shape: pl.BlockSpec(shape, lambda b: (b, 0, 0))
    return [
        full2((_N, _N)),
        per_b((1, 1, _N)),
        per_b((1, 1, _N)),
        per_b((1, 1, _H)),
        full2((2, _H)),
        full2((1, _H)),
        full2((1, _H)),
        full2((_H, _H)),
        full2((1, _H)),
        full2((1, _H)),
        full2((_H, _H)),
        full2((1, _H)),
        full2((2 * _H, _H)),
        full2((1, _H)),
        full2((1, _H)),
        full2((1, 1)),
    ], per_b


def _run_fast(args, interpret=False):
    in_specs, per_b = _in_specs()
    return pl.pallas_call(
        _fast_body,
        grid=(_B,),
        in_specs=in_specs,
        out_specs=[per_b((1, 1, _N)), per_b((1, 1, 128))],
        out_shape=[jax.ShapeDtypeStruct((_B, 1, _N), jnp.float32),
                   jax.ShapeDtypeStruct((_B, 1, 128), jnp.float32)],
        compiler_params=pltpu.CompilerParams(
            dimension_semantics=("parallel",)),
        interpret=interpret,
    )(*args)


def _run_slow(args, interpret=False):
    in_specs, per_b = _in_specs()
    return pl.pallas_call(
        _slow_body,
        grid=(_B,),
        in_specs=in_specs,
        out_specs=per_b((1, 1, _N)),
        out_shape=jax.ShapeDtypeStruct((_B, 1, _N), jnp.float32),
        compiler_params=pltpu.CompilerParams(
            dimension_semantics=("parallel",)),
        interpret=interpret,
    )(*args)


def kernel(adj, ctl_expr, drug_targets, cell_idx, W_ge, b_ge, cell_table,
           attn1, kernel1, bias1, attn2, kernel2, bias2, W1, b1, W2, b2):
    cemb = jnp.take(cell_table, cell_idx, axis=0)
    args = (
        adj,
        ctl_expr.reshape(_B, 1, _N),
        drug_targets.reshape(_B, 1, _N),
        cemb.reshape(_B, 1, _H),
        W_ge,
        b_ge.reshape(1, _H),
        attn1.reshape(1, _H),
        kernel1,
        bias1.reshape(1, _H),
        attn2.reshape(1, _H),
        kernel2,
        bias2.reshape(1, _H),
        W1,
        b1.reshape(1, _H),
        W2.reshape(1, _H),
        b2.reshape(1, 1),
    )
    out_fast, flags = _run_fast(args)
    any_deficient = jnp.sum(flags[:, 0, 0]) > 0.0
    out = jax.lax.cond(
        any_deficient,
        lambda: _run_slow(args),
        lambda: out_fast,
    )
    return out.reshape(_B, _N)
